# baseline (speedup 1.0000x reference)
import functools

import jax
import jax.numpy as jnp
from jax import lax
from jax.experimental import pallas as pl
from jax.experimental.pallas import tpu as pltpu
from jax.experimental.pallas import tpu_sc as plsc

BATCH = 16384
EMBED = 32
LANES = 16
CHUNK = 128
PACK = 4
NROWS = 1000000
NPACK = NROWS // PACK

_info = plsc.get_sparse_core_info()
_NC = _info.num_cores
_NS = _info.num_subcores
NW = _NC * _NS
BPW = BATCH // NW
NCHUNK = BPW // CHUNK
NGROUP = CHUNK // LANES


def _pack(table):
    return jnp.reshape(table, (NPACK, PACK * EMBED))


@functools.partial(
    pl.kernel,
    mesh=plsc.VectorSubcoreMesh(core_axis_name="c", subcore_axis_name="s"),
    out_type=jax.ShapeDtypeStruct((BATCH,), jnp.float32),
    compiler_params=pltpu.CompilerParams(needs_layout_passes=False),
    scratch_types=[
        pltpu.VMEM((BPW,), jnp.int32),
        pltpu.VMEM((BPW,), jnp.int32),
        pltpu.VMEM((BPW,), jnp.int32),
        pltpu.VMEM((BPW,), jnp.int32),
        pltpu.VMEM((CHUNK, PACK * EMBED), jnp.float32),
        pltpu.VMEM((CHUNK, PACK * EMBED), jnp.float32),
        pltpu.VMEM((BPW,), jnp.float32),
        pltpu.SemaphoreType.DMA,
        pltpu.SemaphoreType.DMA,
    ],
)
def _mf_kernel(user_hbm, item_hbm, pu_hbm, pi_hbm, out_hbm,
               idx_u, idx_i, row_u, row_i, rows_u, rows_i, out_v,
               sem_u, sem_i):
    wid = lax.axis_index("s") * _NC + lax.axis_index("c")
    base = wid * BPW

    pltpu.sync_copy(user_hbm.at[pl.ds(base, BPW)], idx_u)
    pltpu.sync_copy(item_hbm.at[pl.ds(base, BPW)], idx_i)

    for k in range(BPW // LANES):
        s = pl.ds(k * LANES, LANES)
        row_u[s] = lax.shift_right_logical(idx_u[s], 2)
        row_i[s] = lax.shift_right_logical(idx_i[s], 2)

    lane_iota = lax.iota(jnp.int32, LANES)

    for c in range(NCHUNK):
        cu = pltpu.async_copy(
            pu_hbm.at[row_u.at[pl.ds(c * CHUNK, CHUNK)]], rows_u, sem_u)
        ci = pltpu.async_copy(
            pi_hbm.at[row_i.at[pl.ds(c * CHUNK, CHUNK)]], rows_i, sem_i)
        cu.wait()
        ci.wait()
        for g in range(NGROUP):
            off = c * CHUNK + g * LANES
            rid = lane_iota + g * LANES
            qu = (idx_u[pl.ds(off, LANES)] & 3) * EMBED
            qi = (idx_i[pl.ds(off, LANES)] & 3) * EMBED
            acc = (plsc.load_gather(rows_u, [rid, qu])
                   * plsc.load_gather(rows_i, [rid, qi]))
            for d in range(1, EMBED):
                acc = acc + (plsc.load_gather(rows_u, [rid, qu + d])
                             * plsc.load_gather(rows_i, [rid, qi + d]))
            out_v[pl.ds(off, LANES)] = acc

    pltpu.sync_copy(out_v, out_hbm.at[pl.ds(base, BPW)])


def kernel(user, item, user_table, item_table):
    return _mf_kernel(user, item, _pack(user_table), _pack(item_table))


# speedup vs baseline: 1.0131x; 1.0131x over previous
import functools

import jax
import jax.numpy as jnp
from jax import lax
from jax.experimental import pallas as pl
from jax.experimental.pallas import tpu as pltpu
from jax.experimental.pallas import tpu_sc as plsc

BATCH = 16384
EMBED = 32
LANES = 16
CHUNK = 128
PACK = 4
NROWS = 1000000
NPACK = NROWS // PACK

_info = plsc.get_sparse_core_info()
_NC = _info.num_cores
_NS = _info.num_subcores
NW = _NC * _NS
BPW = BATCH // NW
NCHUNK = BPW // CHUNK
NGROUP = CHUNK // LANES


QSHIFT = 18
QMASK = (1 << QSHIFT) - 1
NPACK_S = 1 << QSHIFT
_PBLK = 512
_NBLK = NPACK_S // _PBLK
_LASTB = (NROWS + _PBLK - 1) // _PBLK - 1


def _pack_block(t0, t1, t2, t3, o_ref):
    o_ref[:, 0 * EMBED:1 * EMBED] = t0[...].T
    o_ref[:, 1 * EMBED:2 * EMBED] = t1[...].T
    o_ref[:, 2 * EMBED:3 * EMBED] = t2[...].T
    o_ref[:, 3 * EMBED:4 * EMBED] = t3[...].T


def _pack(table):
    spec = lambda j: pl.BlockSpec(
        (EMBED, _PBLK),
        lambda i, j=j: (0, jnp.minimum(i + j * _NBLK, _LASTB)))
    return pl.pallas_call(
        _pack_block,
        grid=(_NBLK,),
        in_specs=[spec(0), spec(1), spec(2), spec(3)],
        out_specs=pl.BlockSpec((_PBLK, PACK * EMBED), lambda i: (i, 0)),
        out_shape=jax.ShapeDtypeStruct((NPACK_S, PACK * EMBED), jnp.float32),
    )(table.T, table.T, table.T, table.T)


@functools.partial(
    pl.kernel,
    mesh=plsc.VectorSubcoreMesh(core_axis_name="c", subcore_axis_name="s"),
    out_type=jax.ShapeDtypeStruct((BATCH,), jnp.float32),
    compiler_params=pltpu.CompilerParams(needs_layout_passes=False),
    scratch_types=[
        pltpu.VMEM((BPW,), jnp.int32),
        pltpu.VMEM((BPW,), jnp.int32),
        pltpu.VMEM((BPW,), jnp.int32),
        pltpu.VMEM((BPW,), jnp.int32),
        pltpu.VMEM((CHUNK, PACK * EMBED), jnp.float32),
        pltpu.VMEM((CHUNK, PACK * EMBED), jnp.float32),
        pltpu.VMEM((BPW,), jnp.float32),
        pltpu.SemaphoreType.DMA,
        pltpu.SemaphoreType.DMA,
    ],
)
def _mf_kernel(user_hbm, item_hbm, pu_hbm, pi_hbm, out_hbm,
               idx_u, idx_i, row_u, row_i, rows_u, rows_i, out_v,
               sem_u, sem_i):
    wid = lax.axis_index("s") * _NC + lax.axis_index("c")
    base = wid * BPW

    pltpu.sync_copy(user_hbm.at[pl.ds(base, BPW)], idx_u)
    pltpu.sync_copy(item_hbm.at[pl.ds(base, BPW)], idx_i)

    for k in range(BPW // LANES):
        s = pl.ds(k * LANES, LANES)
        row_u[s] = idx_u[s] & QMASK
        row_i[s] = idx_i[s] & QMASK

    lane_iota = lax.iota(jnp.int32, LANES)

    for c in range(NCHUNK):
        cu = pltpu.async_copy(
            pu_hbm.at[row_u.at[pl.ds(c * CHUNK, CHUNK)]], rows_u, sem_u)
        ci = pltpu.async_copy(
            pi_hbm.at[row_i.at[pl.ds(c * CHUNK, CHUNK)]], rows_i, sem_i)
        cu.wait()
        ci.wait()
        for g in range(NGROUP):
            off = c * CHUNK + g * LANES
            rid = lane_iota + g * LANES
            qu = lax.shift_right_logical(idx_u[pl.ds(off, LANES)], QSHIFT) * EMBED
            qi = lax.shift_right_logical(idx_i[pl.ds(off, LANES)], QSHIFT) * EMBED
            acc = (plsc.load_gather(rows_u, [rid, qu])
                   * plsc.load_gather(rows_i, [rid, qi]))
            for d in range(1, EMBED):
                acc = acc + (plsc.load_gather(rows_u, [rid, qu + d])
                             * plsc.load_gather(rows_i, [rid, qi + d]))
            out_v[pl.ds(off, LANES)] = acc

    pltpu.sync_copy(out_v, out_hbm.at[pl.ds(base, BPW)])


def kernel(user, item, user_table, item_table):
    return _mf_kernel(user, item, _pack(user_table), _pack(item_table))
